# Initial kernel scaffold; baseline (speedup 1.0000x reference)
import jax
import jax.numpy as jnp
from jax.experimental import pallas as pl


def kernel(points, inverse_indices, W1, b1, gamma1, beta1, W2, b2, gamma2, beta2):
    raise NotImplementedError("write your pallas kernel here")


# speedup vs baseline: 4.6861x; 4.6861x over previous
import functools

import jax
import jax.numpy as jnp
from jax.experimental import pallas as pl

_TN = 1280


def _stats1_kernel(p_ref, w1_ref, sum_ref, sq_ref):
    z1 = jnp.dot(p_ref[...], w1_ref[...], preferred_element_type=jnp.float32)
    s = jnp.sum(z1, axis=0, keepdims=True)
    q = jnp.sum(z1 * z1, axis=0, keepdims=True)

    @pl.when(pl.program_id(0) == 0)
    def _init():
        sum_ref[...] = jnp.broadcast_to(s, sum_ref.shape)
        sq_ref[...] = jnp.broadcast_to(q, sq_ref.shape)

    @pl.when(pl.program_id(0) != 0)
    def _acc():
        sum_ref[...] += jnp.broadcast_to(s, sum_ref.shape)
        sq_ref[...] += jnp.broadcast_to(q, sq_ref.shape)


def _pass2_kernel(p_ref, w1_ref, w2c_ref, scale_ref, shift_ref,
                  idx_ref, idxn_ref, sum_ref, sq_ref, max_ref):
    z1 = jnp.dot(p_ref[...], w1_ref[...], preferred_element_type=jnp.float32)
    h = jnp.maximum(z1 * scale_ref[...] + shift_ref[...], 0.0)
    z2 = jnp.dot(h, w2c_ref[...], preferred_element_type=jnp.float32)
    s = jnp.sum(z2)
    q = jnp.sum(z2 * z2)
    mask = idx_ref[...] != idxn_ref[...]
    m = jnp.max(jnp.where(mask, z2, -jnp.inf))

    @pl.when(pl.program_id(0) == 0)
    def _init():
        sum_ref[...] = jnp.full(sum_ref.shape, s, jnp.float32)
        sq_ref[...] = jnp.full(sq_ref.shape, q, jnp.float32)
        max_ref[...] = jnp.full(max_ref.shape, m, jnp.float32)

    @pl.when(pl.program_id(0) != 0)
    def _acc():
        sum_ref[...] += jnp.full(sum_ref.shape, s, jnp.float32)
        sq_ref[...] += jnp.full(sq_ref.shape, q, jnp.float32)
        max_ref[...] = jnp.maximum(max_ref[...], m)


@functools.partial(jax.jit, static_argnames=("interpret",))
def kernel(points, inverse_indices, W1, b1, gamma1, beta1,
           W2, b2, gamma2, beta2, interpret=False):
    n, d_in = points.shape
    f1 = W1.shape[1]
    tiles = n // _TN
    eps = 1e-5

    sum1, sq1 = pl.pallas_call(
        _stats1_kernel,
        grid=(tiles,),
        in_specs=[
            pl.BlockSpec((_TN, d_in), lambda i: (i, 0)),
            pl.BlockSpec((d_in, f1), lambda i: (0, 0)),
        ],
        out_specs=[
            pl.BlockSpec((8, f1), lambda i: (0, 0)),
            pl.BlockSpec((8, f1), lambda i: (0, 0)),
        ],
        out_shape=[
            jax.ShapeDtypeStruct((8, f1), jnp.float32),
            jax.ShapeDtypeStruct((8, f1), jnp.float32),
        ],
        interpret=interpret,
    )(points, W1)

    mean1 = sum1[0] / n
    var1 = sq1[0] / n - mean1 * mean1
    scale1 = (gamma1 / jnp.sqrt(var1 + eps)).reshape(1, f1)
    shift1 = (beta1 - mean1 * (gamma1 / jnp.sqrt(var1 + eps))).reshape(1, f1)

    idx = inverse_indices.reshape(n, 1)
    idxn = jnp.concatenate(
        [inverse_indices[1:], jnp.full((1,), -1, jnp.int32)]).reshape(n, 1)
    w2c = W2[:, 0:1]

    sum2, sq2, mx = pl.pallas_call(
        _pass2_kernel,
        grid=(tiles,),
        in_specs=[
            pl.BlockSpec((_TN, d_in), lambda i: (i, 0)),
            pl.BlockSpec((d_in, f1), lambda i: (0, 0)),
            pl.BlockSpec((f1, 1), lambda i: (0, 0)),
            pl.BlockSpec((1, f1), lambda i: (0, 0)),
            pl.BlockSpec((1, f1), lambda i: (0, 0)),
            pl.BlockSpec((_TN, 1), lambda i: (i, 0)),
            pl.BlockSpec((_TN, 1), lambda i: (i, 0)),
        ],
        out_specs=[
            pl.BlockSpec((8, 128), lambda i: (0, 0)),
            pl.BlockSpec((8, 128), lambda i: (0, 0)),
            pl.BlockSpec((8, 128), lambda i: (0, 0)),
        ],
        out_shape=[
            jax.ShapeDtypeStruct((8, 128), jnp.float32),
            jax.ShapeDtypeStruct((8, 128), jnp.float32),
            jax.ShapeDtypeStruct((8, 128), jnp.float32),
        ],
        interpret=interpret,
    )(points, W1, w2c, scale1, shift1, idx, idxn)

    mean2 = sum2[0, 0] / n
    var2 = sq2[0, 0] / n - mean2 * mean2
    m = mx[0, 0]
    out = (m - mean2) / jnp.sqrt(var2 + eps) * gamma2[0] + beta2[0]
    return jnp.maximum(out, 0.0)


# speedup vs baseline: 4.6932x; 1.0015x over previous
import functools

import jax
import jax.numpy as jnp
from jax.experimental import pallas as pl

_TN = 1280


def _stats1_kernel(p_ref, w1_ref, sum_ref, sq_ref, z1b_ref):
    z1 = jnp.dot(p_ref[...], w1_ref[...], preferred_element_type=jnp.float32)
    z1b_ref[...] = z1.astype(jnp.bfloat16)
    s = jnp.sum(z1, axis=0, keepdims=True)
    q = jnp.sum(z1 * z1, axis=0, keepdims=True)

    @pl.when(pl.program_id(0) == 0)
    def _init():
        sum_ref[...] = jnp.broadcast_to(s, sum_ref.shape)
        sq_ref[...] = jnp.broadcast_to(q, sq_ref.shape)

    @pl.when(pl.program_id(0) != 0)
    def _acc():
        sum_ref[...] += jnp.broadcast_to(s, sum_ref.shape)
        sq_ref[...] += jnp.broadcast_to(q, sq_ref.shape)


def _pass2_kernel(z1b_ref, w2c_ref, scale_ref, shift_ref,
                  idx_ref, idxn_ref, sum_ref, sq_ref, max_ref):
    z1 = z1b_ref[...].astype(jnp.float32)
    h = jnp.maximum(z1 * scale_ref[...] + shift_ref[...], 0.0)
    z2 = jnp.dot(h, w2c_ref[...], preferred_element_type=jnp.float32)
    s = jnp.sum(z2)
    q = jnp.sum(z2 * z2)
    mask = idx_ref[...] != idxn_ref[...]
    m = jnp.max(jnp.where(mask, z2, -jnp.inf))

    @pl.when(pl.program_id(0) == 0)
    def _init():
        sum_ref[...] = jnp.full(sum_ref.shape, s, jnp.float32)
        sq_ref[...] = jnp.full(sq_ref.shape, q, jnp.float32)
        max_ref[...] = jnp.full(max_ref.shape, m, jnp.float32)

    @pl.when(pl.program_id(0) != 0)
    def _acc():
        sum_ref[...] += jnp.full(sum_ref.shape, s, jnp.float32)
        sq_ref[...] += jnp.full(sq_ref.shape, q, jnp.float32)
        max_ref[...] = jnp.maximum(max_ref[...], m)


@functools.partial(jax.jit, static_argnames=("interpret",))
def kernel(points, inverse_indices, W1, b1, gamma1, beta1,
           W2, b2, gamma2, beta2, interpret=False):
    n, d_in = points.shape
    f1 = W1.shape[1]
    tiles = n // _TN
    eps = 1e-5

    sum1, sq1, z1b = pl.pallas_call(
        _stats1_kernel,
        grid=(tiles,),
        in_specs=[
            pl.BlockSpec((_TN, d_in), lambda i: (i, 0)),
            pl.BlockSpec((d_in, f1), lambda i: (0, 0)),
        ],
        out_specs=[
            pl.BlockSpec((8, f1), lambda i: (0, 0)),
            pl.BlockSpec((8, f1), lambda i: (0, 0)),
            pl.BlockSpec((_TN, f1), lambda i: (i, 0)),
        ],
        out_shape=[
            jax.ShapeDtypeStruct((8, f1), jnp.float32),
            jax.ShapeDtypeStruct((8, f1), jnp.float32),
            jax.ShapeDtypeStruct((n, f1), jnp.bfloat16),
        ],
        interpret=interpret,
    )(points, W1)

    mean1 = sum1[0] / n
    var1 = sq1[0] / n - mean1 * mean1
    scale1 = (gamma1 / jnp.sqrt(var1 + eps)).reshape(1, f1)
    shift1 = (beta1 - mean1 * (gamma1 / jnp.sqrt(var1 + eps))).reshape(1, f1)

    idx = inverse_indices.reshape(n, 1)
    idxn = jnp.concatenate(
        [inverse_indices[1:], jnp.full((1,), -1, jnp.int32)]).reshape(n, 1)
    w2c = W2[:, 0:1]

    sum2, sq2, mx = pl.pallas_call(
        _pass2_kernel,
        grid=(tiles,),
        in_specs=[
            pl.BlockSpec((_TN, f1), lambda i: (i, 0)),
            pl.BlockSpec((f1, 1), lambda i: (0, 0)),
            pl.BlockSpec((1, f1), lambda i: (0, 0)),
            pl.BlockSpec((1, f1), lambda i: (0, 0)),
            pl.BlockSpec((_TN, 1), lambda i: (i, 0)),
            pl.BlockSpec((_TN, 1), lambda i: (i, 0)),
        ],
        out_specs=[
            pl.BlockSpec((8, 128), lambda i: (0, 0)),
            pl.BlockSpec((8, 128), lambda i: (0, 0)),
            pl.BlockSpec((8, 128), lambda i: (0, 0)),
        ],
        out_shape=[
            jax.ShapeDtypeStruct((8, 128), jnp.float32),
            jax.ShapeDtypeStruct((8, 128), jnp.float32),
            jax.ShapeDtypeStruct((8, 128), jnp.float32),
        ],
        interpret=interpret,
    )(z1b, w2c, scale1, shift1, idx, idxn)

    mean2 = sum2[0, 0] / n
    var2 = sq2[0, 0] / n - mean2 * mean2
    m = mx[0, 0]
    out = (m - mean2) / jnp.sqrt(var2 + eps) * gamma2[0] + beta2[0]
    return jnp.maximum(out, 0.0)


# speedup vs baseline: 8.3856x; 1.7867x over previous
import functools

import jax
import jax.numpy as jnp
from jax.experimental import pallas as pl

_TN = 1280


def _stats1_kernel(p_ref, w1_ref, sum_ref, sq_ref, z1t_ref):
    z1 = jnp.dot(p_ref[...], w1_ref[...], preferred_element_type=jnp.float32)
    z1t_ref[...] = z1.T.astype(jnp.bfloat16)
    s = jnp.sum(z1, axis=0, keepdims=True)
    q = jnp.sum(z1 * z1, axis=0, keepdims=True)

    @pl.when(pl.program_id(0) == 0)
    def _init():
        sum_ref[...] = jnp.broadcast_to(s, sum_ref.shape)
        sq_ref[...] = jnp.broadcast_to(q, sq_ref.shape)

    @pl.when(pl.program_id(0) != 0)
    def _acc():
        sum_ref[...] += jnp.broadcast_to(s, sum_ref.shape)
        sq_ref[...] += jnp.broadcast_to(q, sq_ref.shape)


def _pass2_kernel(z1t_ref, w2r_ref, scale_ref, shift_ref,
                  idx_ref, idxn_ref, sum_ref, sq_ref, max_ref):
    z1t = z1t_ref[...].astype(jnp.float32)
    h = jnp.maximum(z1t * scale_ref[...] + shift_ref[...], 0.0)
    z2 = jnp.dot(w2r_ref[...], h, preferred_element_type=jnp.float32)
    s = jnp.sum(z2)
    q = jnp.sum(z2 * z2)
    mask = idx_ref[0] != idxn_ref[0]
    m = jnp.max(jnp.where(mask, z2, -jnp.inf))

    @pl.when(pl.program_id(0) == 0)
    def _init():
        sum_ref[...] = jnp.full(sum_ref.shape, s, jnp.float32)
        sq_ref[...] = jnp.full(sq_ref.shape, q, jnp.float32)
        max_ref[...] = jnp.full(max_ref.shape, m, jnp.float32)

    @pl.when(pl.program_id(0) != 0)
    def _acc():
        sum_ref[...] += jnp.full(sum_ref.shape, s, jnp.float32)
        sq_ref[...] += jnp.full(sq_ref.shape, q, jnp.float32)
        max_ref[...] = jnp.maximum(max_ref[...], m)


@functools.partial(jax.jit, static_argnames=("interpret",))
def kernel(points, inverse_indices, W1, b1, gamma1, beta1,
           W2, b2, gamma2, beta2, interpret=False):
    n, d_in = points.shape
    f1 = W1.shape[1]
    tiles = n // _TN
    eps = 1e-5

    sum1, sq1, z1t = pl.pallas_call(
        _stats1_kernel,
        grid=(tiles,),
        in_specs=[
            pl.BlockSpec((_TN, d_in), lambda i: (i, 0)),
            pl.BlockSpec((d_in, f1), lambda i: (0, 0)),
        ],
        out_specs=[
            pl.BlockSpec((8, f1), lambda i: (0, 0)),
            pl.BlockSpec((8, f1), lambda i: (0, 0)),
            pl.BlockSpec((f1, _TN), lambda i: (0, i)),
        ],
        out_shape=[
            jax.ShapeDtypeStruct((8, f1), jnp.float32),
            jax.ShapeDtypeStruct((8, f1), jnp.float32),
            jax.ShapeDtypeStruct((f1, n), jnp.bfloat16),
        ],
        interpret=interpret,
    )(points, W1)

    mean1 = sum1[0] / n
    var1 = sq1[0] / n - mean1 * mean1
    inv1 = gamma1 / jnp.sqrt(var1 + eps)
    scale1 = inv1.reshape(f1, 1)
    shift1 = (beta1 - mean1 * inv1).reshape(f1, 1)

    idx3 = inverse_indices.reshape(tiles, 1, _TN)
    idxn3 = jnp.concatenate(
        [inverse_indices[1:], jnp.full((1,), -1, jnp.int32)]).reshape(
            tiles, 1, _TN)
    w2r = W2[:, 0].reshape(1, f1)

    sum2, sq2, mx = pl.pallas_call(
        _pass2_kernel,
        grid=(tiles,),
        in_specs=[
            pl.BlockSpec((f1, _TN), lambda i: (0, i)),
            pl.BlockSpec((1, f1), lambda i: (0, 0)),
            pl.BlockSpec((f1, 1), lambda i: (0, 0)),
            pl.BlockSpec((f1, 1), lambda i: (0, 0)),
            pl.BlockSpec((1, 1, _TN), lambda i: (i, 0, 0)),
            pl.BlockSpec((1, 1, _TN), lambda i: (i, 0, 0)),
        ],
        out_specs=[
            pl.BlockSpec((8, 128), lambda i: (0, 0)),
            pl.BlockSpec((8, 128), lambda i: (0, 0)),
            pl.BlockSpec((8, 128), lambda i: (0, 0)),
        ],
        out_shape=[
            jax.ShapeDtypeStruct((8, 128), jnp.float32),
            jax.ShapeDtypeStruct((8, 128), jnp.float32),
            jax.ShapeDtypeStruct((8, 128), jnp.float32),
        ],
        interpret=interpret,
    )(z1t, w2r, scale1, shift1, idx3, idxn3)

    mean2 = sum2[0, 0] / n
    var2 = sq2[0, 0] / n - mean2 * mean2
    m = mx[0, 0]
    out = (m - mean2) / jnp.sqrt(var2 + eps) * gamma2[0] + beta2[0]
    return jnp.maximum(out, 0.0)


# speedup vs baseline: 12.9171x; 1.5404x over previous
import functools

import jax
import jax.numpy as jnp
from jax.experimental import pallas as pl

_TN = 2560


def _stats1_kernel(p_ref, w1_ref, sum_ref, sq_ref, z1t_ref):
    z1 = jnp.dot(p_ref[...], w1_ref[...], preferred_element_type=jnp.float32)
    z1t_ref[0] = z1.T.astype(jnp.bfloat16)
    s = jnp.sum(z1, axis=0, keepdims=True)
    q = jnp.sum(z1 * z1, axis=0, keepdims=True)

    @pl.when(pl.program_id(0) == 0)
    def _init():
        sum_ref[...] = jnp.broadcast_to(s, sum_ref.shape)
        sq_ref[...] = jnp.broadcast_to(q, sq_ref.shape)

    @pl.when(pl.program_id(0) != 0)
    def _acc():
        sum_ref[...] += jnp.broadcast_to(s, sum_ref.shape)
        sq_ref[...] += jnp.broadcast_to(q, sq_ref.shape)


def _pass2_kernel(z1t_ref, w2r_ref, scale_ref, shift_ref,
                  idx_ref, idxn_ref, sum_ref, sq_ref, max_ref):
    z1t = z1t_ref[0].astype(jnp.float32)
    h = jnp.maximum(z1t * scale_ref[...] + shift_ref[...], 0.0)
    z2 = jnp.dot(w2r_ref[...], h, preferred_element_type=jnp.float32)
    s = jnp.sum(z2)
    q = jnp.sum(z2 * z2)
    mask = idx_ref[0] != idxn_ref[0]
    m = jnp.max(jnp.where(mask, z2, -jnp.inf))

    @pl.when(pl.program_id(0) == 0)
    def _init():
        sum_ref[...] = jnp.full(sum_ref.shape, s, jnp.float32)
        sq_ref[...] = jnp.full(sq_ref.shape, q, jnp.float32)
        max_ref[...] = jnp.full(max_ref.shape, m, jnp.float32)

    @pl.when(pl.program_id(0) != 0)
    def _acc():
        sum_ref[...] += jnp.full(sum_ref.shape, s, jnp.float32)
        sq_ref[...] += jnp.full(sq_ref.shape, q, jnp.float32)
        max_ref[...] = jnp.maximum(max_ref[...], m)


@functools.partial(jax.jit, static_argnames=("interpret",))
def kernel(points, inverse_indices, W1, b1, gamma1, beta1,
           W2, b2, gamma2, beta2, interpret=False):
    n, d_in = points.shape
    f1 = W1.shape[1]
    tiles = n // _TN
    eps = 1e-5

    sum1, sq1, z1t = pl.pallas_call(
        _stats1_kernel,
        grid=(tiles,),
        in_specs=[
            pl.BlockSpec((_TN, d_in), lambda i: (i, 0)),
            pl.BlockSpec((d_in, f1), lambda i: (0, 0)),
        ],
        out_specs=[
            pl.BlockSpec((8, f1), lambda i: (0, 0)),
            pl.BlockSpec((8, f1), lambda i: (0, 0)),
            pl.BlockSpec((1, f1, _TN), lambda i: (i, 0, 0)),
        ],
        out_shape=[
            jax.ShapeDtypeStruct((8, f1), jnp.float32),
            jax.ShapeDtypeStruct((8, f1), jnp.float32),
            jax.ShapeDtypeStruct((tiles, f1, _TN), jnp.bfloat16),
        ],
        interpret=interpret,
    )(points, W1)

    mean1 = sum1[0] / n
    var1 = sq1[0] / n - mean1 * mean1
    inv1 = gamma1 / jnp.sqrt(var1 + eps)
    scale1 = inv1.reshape(f1, 1)
    shift1 = (beta1 - mean1 * inv1).reshape(f1, 1)

    idx3 = inverse_indices.reshape(tiles, 1, _TN)
    idxn3 = jnp.concatenate(
        [inverse_indices[1:], jnp.full((1,), -1, jnp.int32)]).reshape(
            tiles, 1, _TN)
    w2r = W2[:, 0].reshape(1, f1)

    sum2, sq2, mx = pl.pallas_call(
        _pass2_kernel,
        grid=(tiles,),
        in_specs=[
            pl.BlockSpec((1, f1, _TN), lambda i: (i, 0, 0)),
            pl.BlockSpec((1, f1), lambda i: (0, 0)),
            pl.BlockSpec((f1, 1), lambda i: (0, 0)),
            pl.BlockSpec((f1, 1), lambda i: (0, 0)),
            pl.BlockSpec((1, 1, _TN), lambda i: (i, 0, 0)),
            pl.BlockSpec((1, 1, _TN), lambda i: (i, 0, 0)),
        ],
        out_specs=[
            pl.BlockSpec((8, 128), lambda i: (0, 0)),
            pl.BlockSpec((8, 128), lambda i: (0, 0)),
            pl.BlockSpec((8, 128), lambda i: (0, 0)),
        ],
        out_shape=[
            jax.ShapeDtypeStruct((8, 128), jnp.float32),
            jax.ShapeDtypeStruct((8, 128), jnp.float32),
            jax.ShapeDtypeStruct((8, 128), jnp.float32),
        ],
        interpret=interpret,
    )(z1t, w2r, scale1, shift1, idx3, idxn3)

    mean2 = sum2[0, 0] / n
    var2 = sq2[0, 0] / n - mean2 * mean2
    m = mx[0, 0]
    out = (m - mean2) / jnp.sqrt(var2 + eps) * gamma2[0] + beta2[0]
    return jnp.maximum(out, 0.0)


# speedup vs baseline: 19.8668x; 1.5380x over previous
import functools

import jax
import jax.numpy as jnp
from jax.experimental import pallas as pl

_TN = 6400


def _stats1_kernel(p_ref, w1_ref, sum_ref, sq_ref, z1t_ref):
    z1 = jnp.dot(p_ref[...], w1_ref[...], preferred_element_type=jnp.float32)
    z1t_ref[0] = z1.astype(jnp.bfloat16).T
    s = jnp.sum(z1, axis=0, keepdims=True)
    q = jnp.sum(z1 * z1, axis=0, keepdims=True)

    @pl.when(pl.program_id(0) == 0)
    def _init():
        sum_ref[...] = jnp.broadcast_to(s, sum_ref.shape)
        sq_ref[...] = jnp.broadcast_to(q, sq_ref.shape)

    @pl.when(pl.program_id(0) != 0)
    def _acc():
        sum_ref[...] += jnp.broadcast_to(s, sum_ref.shape)
        sq_ref[...] += jnp.broadcast_to(q, sq_ref.shape)


def _pass2_kernel(z1t_ref, w2r_ref, scale_ref, shift_ref,
                  idx_ref, idxn_ref, sum_ref, sq_ref, max_ref):
    z1t = z1t_ref[0].astype(jnp.float32)
    h = jnp.maximum(z1t * scale_ref[...] + shift_ref[...], 0.0)
    z2 = jnp.dot(w2r_ref[...], h, preferred_element_type=jnp.float32)
    s = jnp.sum(z2)
    q = jnp.sum(z2 * z2)
    mask = idx_ref[0] != idxn_ref[0]
    m = jnp.max(jnp.where(mask, z2, -jnp.inf))

    @pl.when(pl.program_id(0) == 0)
    def _init():
        sum_ref[...] = jnp.full(sum_ref.shape, s, jnp.float32)
        sq_ref[...] = jnp.full(sq_ref.shape, q, jnp.float32)
        max_ref[...] = jnp.full(max_ref.shape, m, jnp.float32)

    @pl.when(pl.program_id(0) != 0)
    def _acc():
        sum_ref[...] += jnp.full(sum_ref.shape, s, jnp.float32)
        sq_ref[...] += jnp.full(sq_ref.shape, q, jnp.float32)
        max_ref[...] = jnp.maximum(max_ref[...], m)


@functools.partial(jax.jit, static_argnames=("interpret",))
def kernel(points, inverse_indices, W1, b1, gamma1, beta1,
           W2, b2, gamma2, beta2, interpret=False):
    n, d_in = points.shape
    f1 = W1.shape[1]
    tiles = n // _TN
    eps = 1e-5

    sum1, sq1, z1t = pl.pallas_call(
        _stats1_kernel,
        grid=(tiles,),
        in_specs=[
            pl.BlockSpec((_TN, d_in), lambda i: (i, 0)),
            pl.BlockSpec((d_in, f1), lambda i: (0, 0)),
        ],
        out_specs=[
            pl.BlockSpec((8, f1), lambda i: (0, 0)),
            pl.BlockSpec((8, f1), lambda i: (0, 0)),
            pl.BlockSpec((1, f1, _TN), lambda i: (i, 0, 0)),
        ],
        out_shape=[
            jax.ShapeDtypeStruct((8, f1), jnp.float32),
            jax.ShapeDtypeStruct((8, f1), jnp.float32),
            jax.ShapeDtypeStruct((tiles, f1, _TN), jnp.bfloat16),
        ],
        interpret=interpret,
    )(points, W1)

    mean1 = sum1[0] / n
    var1 = sq1[0] / n - mean1 * mean1
    inv1 = gamma1 / jnp.sqrt(var1 + eps)
    scale1 = inv1.reshape(f1, 1)
    shift1 = (beta1 - mean1 * inv1).reshape(f1, 1)

    idx3 = inverse_indices.reshape(tiles, 1, _TN)
    idxn3 = jnp.concatenate(
        [inverse_indices[1:], jnp.full((1,), -1, jnp.int32)]).reshape(
            tiles, 1, _TN)
    w2r = W2[:, 0].reshape(1, f1)

    sum2, sq2, mx = pl.pallas_call(
        _pass2_kernel,
        grid=(tiles,),
        in_specs=[
            pl.BlockSpec((1, f1, _TN), lambda i: (i, 0, 0)),
            pl.BlockSpec((1, f1), lambda i: (0, 0)),
            pl.BlockSpec((f1, 1), lambda i: (0, 0)),
            pl.BlockSpec((f1, 1), lambda i: (0, 0)),
            pl.BlockSpec((1, 1, _TN), lambda i: (i, 0, 0)),
            pl.BlockSpec((1, 1, _TN), lambda i: (i, 0, 0)),
        ],
        out_specs=[
            pl.BlockSpec((8, 128), lambda i: (0, 0)),
            pl.BlockSpec((8, 128), lambda i: (0, 0)),
            pl.BlockSpec((8, 128), lambda i: (0, 0)),
        ],
        out_shape=[
            jax.ShapeDtypeStruct((8, 128), jnp.float32),
            jax.ShapeDtypeStruct((8, 128), jnp.float32),
            jax.ShapeDtypeStruct((8, 128), jnp.float32),
        ],
        interpret=interpret,
    )(z1t, w2r, scale1, shift1, idx3, idxn3)

    mean2 = sum2[0, 0] / n
    var2 = sq2[0, 0] / n - mean2 * mean2
    m = mx[0, 0]
    out = (m - mean2) / jnp.sqrt(var2 + eps) * gamma2[0] + beta2[0]
    return jnp.maximum(out, 0.0)


# speedup vs baseline: 23.5246x; 1.1841x over previous
import functools

import jax
import jax.numpy as jnp
from jax.experimental import pallas as pl

_TN = 12800


def _stats1_kernel(p_ref, w1_ref, sum_ref, sq_ref, z1t_ref):
    z1 = jnp.dot(p_ref[...], w1_ref[...], preferred_element_type=jnp.float32)
    z1t_ref[0] = z1.astype(jnp.bfloat16).T
    s = jnp.sum(z1, axis=0, keepdims=True)
    q = jnp.sum(z1 * z1, axis=0, keepdims=True)

    @pl.when(pl.program_id(0) == 0)
    def _init():
        sum_ref[...] = jnp.broadcast_to(s, sum_ref.shape)
        sq_ref[...] = jnp.broadcast_to(q, sq_ref.shape)

    @pl.when(pl.program_id(0) != 0)
    def _acc():
        sum_ref[...] += jnp.broadcast_to(s, sum_ref.shape)
        sq_ref[...] += jnp.broadcast_to(q, sq_ref.shape)


def _pass2_kernel(z1t_ref, w2r_ref, scale_ref, shift_ref,
                  idx_ref, idxn_ref, sum_ref, sq_ref, max_ref):
    z1t = z1t_ref[0].astype(jnp.float32)
    h = jnp.maximum(z1t * scale_ref[...] + shift_ref[...], 0.0)
    z2 = jnp.dot(w2r_ref[...], h, preferred_element_type=jnp.float32)
    s = jnp.sum(z2)
    q = jnp.sum(z2 * z2)
    mask = idx_ref[0] != idxn_ref[0]
    m = jnp.max(jnp.where(mask, z2, -jnp.inf))

    @pl.when(pl.program_id(0) == 0)
    def _init():
        sum_ref[...] = jnp.full(sum_ref.shape, s, jnp.float32)
        sq_ref[...] = jnp.full(sq_ref.shape, q, jnp.float32)
        max_ref[...] = jnp.full(max_ref.shape, m, jnp.float32)

    @pl.when(pl.program_id(0) != 0)
    def _acc():
        sum_ref[...] += jnp.full(sum_ref.shape, s, jnp.float32)
        sq_ref[...] += jnp.full(sq_ref.shape, q, jnp.float32)
        max_ref[...] = jnp.maximum(max_ref[...], m)


@functools.partial(jax.jit, static_argnames=("interpret",))
def kernel(points, inverse_indices, W1, b1, gamma1, beta1,
           W2, b2, gamma2, beta2, interpret=False):
    n, d_in = points.shape
    f1 = W1.shape[1]
    tiles = n // _TN
    eps = 1e-5

    sum1, sq1, z1t = pl.pallas_call(
        _stats1_kernel,
        grid=(tiles,),
        in_specs=[
            pl.BlockSpec((_TN, d_in), lambda i: (i, 0)),
            pl.BlockSpec((d_in, f1), lambda i: (0, 0)),
        ],
        out_specs=[
            pl.BlockSpec((8, f1), lambda i: (0, 0)),
            pl.BlockSpec((8, f1), lambda i: (0, 0)),
            pl.BlockSpec((1, f1, _TN), lambda i: (i, 0, 0)),
        ],
        out_shape=[
            jax.ShapeDtypeStruct((8, f1), jnp.float32),
            jax.ShapeDtypeStruct((8, f1), jnp.float32),
            jax.ShapeDtypeStruct((tiles, f1, _TN), jnp.bfloat16),
        ],
        interpret=interpret,
    )(points, W1)

    mean1 = sum1[0] / n
    var1 = sq1[0] / n - mean1 * mean1
    inv1 = gamma1 / jnp.sqrt(var1 + eps)
    scale1 = inv1.reshape(f1, 1)
    shift1 = (beta1 - mean1 * inv1).reshape(f1, 1)

    idx3 = inverse_indices.reshape(tiles, 1, _TN)
    idxn3 = jnp.concatenate(
        [inverse_indices[1:], jnp.full((1,), -1, jnp.int32)]).reshape(
            tiles, 1, _TN)
    w2r = W2[:, 0].reshape(1, f1)

    sum2, sq2, mx = pl.pallas_call(
        _pass2_kernel,
        grid=(tiles,),
        in_specs=[
            pl.BlockSpec((1, f1, _TN), lambda i: (i, 0, 0)),
            pl.BlockSpec((1, f1), lambda i: (0, 0)),
            pl.BlockSpec((f1, 1), lambda i: (0, 0)),
            pl.BlockSpec((f1, 1), lambda i: (0, 0)),
            pl.BlockSpec((1, 1, _TN), lambda i: (i, 0, 0)),
            pl.BlockSpec((1, 1, _TN), lambda i: (i, 0, 0)),
        ],
        out_specs=[
            pl.BlockSpec((8, 128), lambda i: (0, 0)),
            pl.BlockSpec((8, 128), lambda i: (0, 0)),
            pl.BlockSpec((8, 128), lambda i: (0, 0)),
        ],
        out_shape=[
            jax.ShapeDtypeStruct((8, 128), jnp.float32),
            jax.ShapeDtypeStruct((8, 128), jnp.float32),
            jax.ShapeDtypeStruct((8, 128), jnp.float32),
        ],
        interpret=interpret,
    )(z1t, w2r, scale1, shift1, idx3, idxn3)

    mean2 = sum2[0, 0] / n
    var2 = sq2[0, 0] / n - mean2 * mean2
    m = mx[0, 0]
    out = (m - mean2) / jnp.sqrt(var2 + eps) * gamma2[0] + beta2[0]
    return jnp.maximum(out, 0.0)
